# Initial kernel scaffold; baseline (speedup 1.0000x reference)
import jax
import jax.numpy as jnp
from jax.experimental import pallas as pl


def kernel(input_ids, segment_ids, token_table, pos_table, seg_table, gamma, beta):
    raise NotImplementedError("write your pallas kernel here")


# speedup vs baseline: 2.8008x; 2.8008x over previous
import functools

import jax
import jax.numpy as jnp
from jax import lax
from jax.experimental import pallas as pl
from jax.experimental.pallas import tpu as pltpu
from jax.experimental.pallas import tpu_sc as plsc

_NC = 2
_NS = 16
_NW = _NC * _NS


def _sc_gather(table, ids, *, chunk):
    bs = ids.shape[0]
    d = table.shape[1]
    per_w = bs // _NW
    nchunks = per_w // chunk
    mesh = plsc.VectorSubcoreMesh(core_axis_name="c", subcore_axis_name="s")

    @functools.partial(
        pl.kernel,
        out_type=jax.ShapeDtypeStruct((bs, d), jnp.float32),
        mesh=mesh,
        scratch_types=[
            pltpu.VMEM((chunk,), jnp.int32),
            pltpu.VMEM((chunk,), jnp.int32),
            pltpu.VMEM((chunk, d), jnp.float32),
            pltpu.VMEM((chunk, d), jnp.float32),
            pltpu.SemaphoreType.DMA,
            pltpu.SemaphoreType.DMA,
            pltpu.SemaphoreType.DMA,
            pltpu.SemaphoreType.DMA,
        ],
    )
    def gather_kernel(table_hbm, ids_hbm, out_hbm,
                      idx0, idx1, buf0, buf1, g0, g1, o0, o1):
        wid = lax.axis_index("s") * _NC + lax.axis_index("c")
        base = wid * per_w
        idx = (idx0, idx1)
        buf = (buf0, buf1)
        gsem = (g0, g1)
        osem = (o0, o1)

        def start_gather(c, p):
            pltpu.sync_copy(ids_hbm.at[pl.ds(base + c * chunk, chunk)], idx[p])
            pltpu.async_copy(table_hbm.at[idx[p]], buf[p], gsem[p])

        def start_out(c, p):
            pltpu.async_copy(
                buf[p], out_hbm.at[pl.ds(base + c * chunk, chunk), :], osem[p])

        start_gather(0, 0)
        for c in range(1, nchunks):
            p = c % 2
            q = (c - 1) % 2
            if c >= 2:
                pltpu.make_async_copy(
                    buf[p], out_hbm.at[pl.ds(0, chunk), :], osem[p]).wait()
            start_gather(c, p)
            pltpu.make_async_copy(
                table_hbm.at[idx[q]], buf[q], gsem[q]).wait()
            start_out(c - 1, q)
        last = nchunks - 1
        pltpu.make_async_copy(
            table_hbm.at[idx[last % 2]], buf[last % 2], gsem[last % 2]).wait()
        start_out(last, last % 2)
        pltpu.make_async_copy(
            buf[0], out_hbm.at[pl.ds(0, chunk), :], osem[0]).wait()
        pltpu.make_async_copy(
            buf[1], out_hbm.at[pl.ds(0, chunk), :], osem[1]).wait()

    return gather_kernel(table, ids)


def _ln_body(seg_ref, g_ref, pos_ref, segtab_ref, gamma_ref, beta_ref, o_ref):
    x = g_ref[...] + pos_ref[...]
    w = (seg_ref[...] == 1).astype(jnp.float32)
    s0 = segtab_ref[0:1, :]
    s1 = segtab_ref[1:2, :]
    x = x + s0 + w * (s1 - s0)
    mean = jnp.mean(x, axis=-1, keepdims=True)
    xc = x - mean
    var = jnp.mean(xc * xc, axis=-1, keepdims=True)
    y = xc * lax.rsqrt(var + 1e-12)
    o_ref[...] = y * gamma_ref[...] + beta_ref[...]


def _tc_ln(gathered, seg_ids, pos_table, seg_table, gamma, beta, *, tile):
    bs, d = gathered.shape
    s = pos_table.shape[0]
    grid = (bs // tile,)
    nseg = seg_table.shape[0]
    return pl.pallas_call(
        _ln_body,
        grid=grid,
        in_specs=[
            pl.BlockSpec((tile, 1), lambda i: (i, 0)),
            pl.BlockSpec((tile, d), lambda i: (i, 0)),
            pl.BlockSpec((s, d), lambda i: (0, 0)),
            pl.BlockSpec((nseg, d), lambda i: (0, 0)),
            pl.BlockSpec((1, d), lambda i: (0, 0)),
            pl.BlockSpec((1, d), lambda i: (0, 0)),
        ],
        out_specs=pl.BlockSpec((tile, d), lambda i: (i, 0)),
        out_shape=jax.ShapeDtypeStruct((bs, d), jnp.float32),
    )(seg_ids, gathered, pos_table, seg_table, gamma, beta)


def kernel(input_ids, segment_ids, token_table, pos_table, seg_table, gamma, beta):
    b, s = input_ids.shape
    d = token_table.shape[1]
    ids = input_ids.reshape(-1)
    gathered = _sc_gather(token_table, ids, chunk=64)
    out = _tc_ln(
        gathered,
        segment_ids.reshape(-1, 1),
        pos_table,
        seg_table,
        gamma.reshape(1, -1),
        beta.reshape(1, -1),
        tile=s,
    )
    return out.reshape(b, s, d)
